# Initial kernel scaffold; baseline (speedup 1.0000x reference)
import jax
import jax.numpy as jnp
from jax.experimental import pallas as pl


def kernel(node_embeddings, Z_embeddings, neighbour_distances, edge_embedding, graph, bessel_freqs, mlp_w1, mlp_b1, mlp_w2, mlp_b2, mlp_w3, mlp_b3, w_pre, w_post, w_self):
    raise NotImplementedError("write your pallas kernel here")


# speedup vs baseline: 1.4354x; 1.4354x over previous
import functools

import jax
import jax.numpy as jnp
from jax import lax
from jax.experimental import pallas as pl
from jax.experimental.pallas import tpu as pltpu
from jax.experimental.pallas import tpu_sc as plsc

_CUTOFF = 5.0


def _silu(x):
    return x / (1.0 + jnp.exp(-x))


def _xlin_body(x_ref, w_ref, o_ref):
    o_ref[...] = jnp.dot(x_ref[...], w_ref[...],
                         preferred_element_type=jnp.float32)


def _tc_xlin(node, w_pre, bm):
    n, d = node.shape
    return pl.pallas_call(
        _xlin_body,
        grid=(n // bm,),
        in_specs=[
            pl.BlockSpec((bm, d), lambda i: (i, 0)),
            pl.BlockSpec((d, d), lambda i: (0, 0)),
        ],
        out_specs=pl.BlockSpec((bm, d), lambda i: (i, 0)),
        out_shape=jax.ShapeDtypeStruct((n, d), jnp.float32),
    )(node, w_pre)


def _wgen_body(r_ref, ee_ref, fr_ref, w1_ref, b1_ref, w2_ref, b2_ref,
               w3_ref, b3_ref, o_ref):
    r = r_ref[...]
    bessel = jnp.sqrt(2.0 / _CUTOFF) * jnp.sin(r * fr_ref[...]) / r
    h = _silu(jnp.dot(bessel, w1_ref[...],
                      preferred_element_type=jnp.float32) + b1_ref[...])
    h = _silu(jnp.dot(h, w2_ref[...],
                      preferred_element_type=jnp.float32) + b2_ref[...])
    w = jnp.dot(h, w3_ref[...],
                preferred_element_type=jnp.float32) + b3_ref[...]
    u = r / _CUTOFF
    u2 = u * u
    u3 = u2 * u
    u6 = u3 * u3
    env = 1.0 - 28.0 * u6 + 48.0 * u6 * u - 21.0 * u6 * u2
    env = jnp.where(u < 1.0, env, 0.0)
    o_ref[...] = w * (env * ee_ref[...])


def _tc_wgen(dist, ee, freqs, w1, b1, w2, b2, w3, b3, be):
    e = dist.shape[0]
    rb = freqs.shape[0]
    d = w3.shape[1]
    r2 = dist.reshape(e, 1)
    fr = freqs.reshape(1, rb)
    grid = (e // be,)
    full = lambda i: (0, 0)
    return pl.pallas_call(
        _wgen_body,
        grid=grid,
        in_specs=[
            pl.BlockSpec((be, 1), lambda i: (i, 0)),
            pl.BlockSpec((be, 1), lambda i: (i, 0)),
            pl.BlockSpec((1, rb), full),
            pl.BlockSpec((rb, rb), full),
            pl.BlockSpec((1, rb), full),
            pl.BlockSpec((rb, rb), full),
            pl.BlockSpec((1, rb), full),
            pl.BlockSpec((rb, d), full),
            pl.BlockSpec((1, d), full),
        ],
        out_specs=pl.BlockSpec((be, d), lambda i: (i, 0)),
        out_shape=jax.ShapeDtypeStruct((e, d), jnp.float32),
    )(r2, ee, fr, w1, b1.reshape(1, rb), w2, b2.reshape(1, rb),
      w3, b3.reshape(1, d))


def _sc_message(xlin, wfull, nbr, cent, n_nodes):
    e, d = wfull.shape
    ncores, nsub = 2, 16
    nworkers = ncores * nsub
    epw = e // nworkers
    chunk = 80
    nchunks = epw // chunk
    rows_per_tile = (n_nodes // nsub) // 8 * 8
    tail_rows = n_nodes - rows_per_tile * nsub
    nz_full = rows_per_tile // chunk
    nz_rem = rows_per_tile - nz_full * chunk

    mesh = plsc.VectorSubcoreMesh(core_axis_name="c", subcore_axis_name="s")

    @functools.partial(
        pl.kernel,
        mesh=mesh,
        out_type=jax.ShapeDtypeStruct((ncores, n_nodes, d), jnp.float32),
        scratch_types=[
            pltpu.VMEM((chunk,), jnp.int32),
            pltpu.VMEM((chunk,), jnp.int32),
            pltpu.VMEM((chunk, d), jnp.float32),
            pltpu.VMEM((chunk, d), jnp.float32),
            pltpu.VMEM_SHARED((n_nodes, d), jnp.float32),
            pltpu.SemaphoreType.DMA,
        ],
    )
    def k(xlin_hbm, w_hbm, nbr_hbm, cent_hbm, out_hbm,
          nbr_v, cent_v, rows_v, msg_v, acc, sem):
        c = lax.axis_index("c")
        s = lax.axis_index("s")
        wid = c * nsub + s

        zero = jnp.zeros((16,), jnp.float32)

        def zrow(i, carry):
            for j in range(d // 16):
                msg_v[i, pl.ds(j * 16, 16)] = zero
            return carry

        lax.fori_loop(0, chunk, zrow, 0)
        base_row = s * rows_per_tile
        for rblk in range(nz_full):
            pltpu.sync_copy(msg_v, acc.at[pl.ds(base_row + rblk * chunk,
                                                chunk)])
        if nz_rem:
            pltpu.sync_copy(msg_v.at[pl.ds(0, nz_rem)],
                            acc.at[pl.ds(base_row + nz_full * chunk, nz_rem)])

        @pl.when(s == nsub - 1)
        def _zero_tail():
            pltpu.sync_copy(msg_v.at[pl.ds(0, tail_rows)],
                            acc.at[pl.ds(rows_per_tile * nsub, tail_rows)])

        plsc.subcore_barrier()

        ebase = wid * epw

        def body(i, carry):
            off = ebase + i * chunk
            pltpu.sync_copy(nbr_hbm.at[pl.ds(off, chunk)], nbr_v)
            pltpu.sync_copy(cent_hbm.at[pl.ds(off, chunk)], cent_v)
            pltpu.async_copy(xlin_hbm.at[nbr_v], rows_v, sem).wait()
            pltpu.sync_copy(w_hbm.at[pl.ds(off, chunk)], msg_v)

            def mrow(r, carry2):
                for j in range(d // 16):
                    sl = pl.ds(j * 16, 16)
                    msg_v[r, sl] = msg_v[r, sl] * rows_v[r, sl]
                return carry2

            lax.fori_loop(0, chunk, mrow, 0)
            pltpu.sync_copy(msg_v, acc.at[cent_v], add=True)
            return carry

        lax.fori_loop(0, nchunks, body, 0)
        plsc.subcore_barrier()
        pltpu.sync_copy(acc.at[pl.ds(base_row, rows_per_tile)],
                        out_hbm.at[c, pl.ds(base_row, rows_per_tile)])

        @pl.when(s == nsub - 1)
        def _write_tail():
            pltpu.sync_copy(acc.at[pl.ds(rows_per_tile * nsub, tail_rows)],
                            out_hbm.at[c, pl.ds(rows_per_tile * nsub,
                                                tail_rows)])

    return k(xlin, wfull, nbr, cent)


def _final_body(p0_ref, p1_ref, nb_ref, wpost_ref, wself_ref, o_ref):
    t = p0_ref[...] + p1_ref[...]
    y = (jnp.dot(t, wpost_ref[...], preferred_element_type=jnp.float32)
         + jnp.dot(nb_ref[...], wself_ref[...],
                   preferred_element_type=jnp.float32))
    o_ref[...] = _silu(y)


def _tc_final(p0, p1, node, w_post, w_self, bm):
    n, d = node.shape
    full = lambda i: (0, 0)
    blk = pl.BlockSpec((bm, d), lambda i: (i, 0))
    return pl.pallas_call(
        _final_body,
        grid=(n // bm,),
        in_specs=[blk, blk, blk,
                  pl.BlockSpec((d, d), full),
                  pl.BlockSpec((d, d), full)],
        out_specs=blk,
        out_shape=jax.ShapeDtypeStruct((n, d), jnp.float32),
    )(p0, p1, node, w_post, w_self)


def kernel(node_embeddings, Z_embeddings, neighbour_distances, edge_embedding,
           graph, bessel_freqs, mlp_w1, mlp_b1, mlp_w2, mlp_b2, mlp_w3,
           mlp_b3, w_pre, w_post, w_self):
    n, d = node_embeddings.shape
    cent = graph[0]
    nbr = graph[1]
    x_lin = _tc_xlin(node_embeddings, w_pre, bm=400)
    wfull = _tc_wgen(neighbour_distances, edge_embedding, bessel_freqs,
                     mlp_w1, mlp_b1, mlp_w2, mlp_b2, mlp_w3, mlp_b3, be=4000)
    partials = _sc_message(x_lin, wfull, nbr, cent, n)
    return _tc_final(partials[0], partials[1], node_embeddings,
                     w_post, w_self, bm=400)


# speedup vs baseline: 2.0217x; 1.4084x over previous
import functools

import jax
import jax.numpy as jnp
from jax import lax
from jax.experimental import pallas as pl
from jax.experimental.pallas import tpu as pltpu
from jax.experimental.pallas import tpu_sc as plsc

_CUTOFF = 5.0


def _silu(x):
    return x / (1.0 + jnp.exp(-x))


def _xlin_body(x_ref, w_ref, o_ref):
    o_ref[...] = jnp.dot(x_ref[...], w_ref[...],
                         preferred_element_type=jnp.float32)


def _tc_xlin(node, w_pre, bm):
    n, d = node.shape
    return pl.pallas_call(
        _xlin_body,
        grid=(n // bm,),
        in_specs=[
            pl.BlockSpec((bm, d), lambda i: (i, 0)),
            pl.BlockSpec((d, d), lambda i: (0, 0)),
        ],
        out_specs=pl.BlockSpec((bm, d), lambda i: (i, 0)),
        out_shape=jax.ShapeDtypeStruct((n, d), jnp.float32),
    )(node, w_pre)


def _wgen_body(bm, rb, r_ref, ee_ref, fr_ref, w1t_ref, b1_ref, w2t_ref,
               b2_ref, w3_ref, b3_ref, o_ref):
    r = r_ref[0]
    r3 = r[:, None, :]
    fr = fr_ref[...][None, :, None]
    bes = jnp.sin(r3 * fr) * (jnp.sqrt(2.0 / _CUTOFF) / r3)
    u = r / _CUTOFF
    u2 = u * u
    u6 = u2 * u2 * u2
    env = 1.0 - 28.0 * u6 + 48.0 * u6 * u - 21.0 * u6 * u2
    env = jnp.where(u < 1.0, env, 0.0)
    scale = env * ee_ref[0]
    w1t = w1t_ref[...]
    w2t = w2t_ref[...]
    w3 = w3_ref[...]
    b1 = b1_ref[...]
    b2 = b2_ref[...]
    b3 = b3_ref[...]
    dn = (((1,), (0,)), ((), ()))
    dnt = (((0,), (0,)), ((), ()))
    for i in range(bm):
        x = bes[i]
        h = _silu(lax.dot_general(w1t, x, dn,
                                  preferred_element_type=jnp.float32) + b1)
        h = _silu(lax.dot_general(w2t, h, dn,
                                  preferred_element_type=jnp.float32) + b2)
        g = h * scale[i][None, :]
        sc1 = scale[i][None, :]
        o_ref[0, i] = (lax.dot_general(g, w3, dnt,
                                       preferred_element_type=jnp.float32)
                       + lax.dot_general(sc1, b3, dnt,
                                         preferred_element_type=jnp.float32))


def _tc_wgen(dist, ee, freqs, w1, b1, w2, b2, w3, b3, bm):
    e = dist.shape[0]
    rb = freqs.shape[0]
    d = w3.shape[1]
    nl = 128
    nblk = e // (nl * bm)
    r2 = dist.reshape(nblk, bm, nl)
    ee2 = ee.reshape(nblk, bm, nl)
    full = lambda i: (0, 0)
    w4d = pl.pallas_call(
        functools.partial(_wgen_body, bm, rb),
        grid=(nblk,),
        in_specs=[
            pl.BlockSpec((1, bm, nl), lambda i: (i, 0, 0)),
            pl.BlockSpec((1, bm, nl), lambda i: (i, 0, 0)),
            pl.BlockSpec((rb,), lambda i: (0,)),
            pl.BlockSpec((rb, rb), full),
            pl.BlockSpec((rb, 1), full),
            pl.BlockSpec((rb, rb), full),
            pl.BlockSpec((rb, 1), full),
            pl.BlockSpec((rb, d), full),
            pl.BlockSpec((1, d), full),
        ],
        out_specs=pl.BlockSpec((1, bm, nl, d), lambda i: (i, 0, 0, 0)),
        out_shape=jax.ShapeDtypeStruct((nblk, bm, nl, d), jnp.float32),
    )(r2, ee2, freqs, w1.T, b1.reshape(rb, 1), w2.T, b2.reshape(rb, 1),
      w3, b3.reshape(1, d))
    return w4d.reshape(e, d)


def _sc_message(xlin, wfull, nbr, cent, n_nodes):
    e, d = wfull.shape
    ncores, nsub = 2, 16
    nworkers = ncores * nsub
    epw = e // nworkers
    chunk = 80
    nchunks = epw // chunk
    rows_per_tile = (n_nodes // nsub) // 8 * 8
    tail_rows = n_nodes - rows_per_tile * nsub
    nz_full = rows_per_tile // chunk
    nz_rem = rows_per_tile - nz_full * chunk

    mesh = plsc.VectorSubcoreMesh(core_axis_name="c", subcore_axis_name="s")

    @functools.partial(
        pl.kernel,
        mesh=mesh,
        out_type=jax.ShapeDtypeStruct((ncores, n_nodes, d), jnp.float32),
        scratch_types=[
            pltpu.VMEM((chunk,), jnp.int32),
            pltpu.VMEM((chunk,), jnp.int32),
            pltpu.VMEM((chunk, d), jnp.float32),
            pltpu.VMEM((chunk, d), jnp.float32),
            pltpu.VMEM_SHARED((n_nodes, d), jnp.float32),
            pltpu.SemaphoreType.DMA,
        ],
    )
    def k(xlin_hbm, w_hbm, nbr_hbm, cent_hbm, out_hbm,
          nbr_v, cent_v, rows_v, msg_v, acc, sem):
        c = lax.axis_index("c")
        s = lax.axis_index("s")
        wid = c * nsub + s

        zero = jnp.zeros((16,), jnp.float32)

        def zrow(i, carry):
            for j in range(d // 16):
                msg_v[i, pl.ds(j * 16, 16)] = zero
            return carry

        lax.fori_loop(0, chunk, zrow, 0)
        base_row = s * rows_per_tile
        for rblk in range(nz_full):
            pltpu.sync_copy(msg_v, acc.at[pl.ds(base_row + rblk * chunk,
                                                chunk)])
        if nz_rem:
            pltpu.sync_copy(msg_v.at[pl.ds(0, nz_rem)],
                            acc.at[pl.ds(base_row + nz_full * chunk, nz_rem)])

        @pl.when(s == nsub - 1)
        def _zero_tail():
            pltpu.sync_copy(msg_v.at[pl.ds(0, tail_rows)],
                            acc.at[pl.ds(rows_per_tile * nsub, tail_rows)])

        plsc.subcore_barrier()

        ebase = wid * epw

        def body(i, carry):
            off = ebase + i * chunk
            pltpu.sync_copy(nbr_hbm.at[pl.ds(off, chunk)], nbr_v)
            pltpu.sync_copy(cent_hbm.at[pl.ds(off, chunk)], cent_v)
            pltpu.async_copy(xlin_hbm.at[nbr_v], rows_v, sem).wait()
            pltpu.sync_copy(w_hbm.at[pl.ds(off, chunk)], msg_v)

            def mrow(r, carry2):
                for j in range(d // 16):
                    sl = pl.ds(j * 16, 16)
                    msg_v[r, sl] = msg_v[r, sl] * rows_v[r, sl]
                return carry2

            lax.fori_loop(0, chunk, mrow, 0)
            pltpu.sync_copy(msg_v, acc.at[cent_v], add=True)
            return carry

        lax.fori_loop(0, nchunks, body, 0)
        plsc.subcore_barrier()
        pltpu.sync_copy(acc.at[pl.ds(base_row, rows_per_tile)],
                        out_hbm.at[c, pl.ds(base_row, rows_per_tile)])

        @pl.when(s == nsub - 1)
        def _write_tail():
            pltpu.sync_copy(acc.at[pl.ds(rows_per_tile * nsub, tail_rows)],
                            out_hbm.at[c, pl.ds(rows_per_tile * nsub,
                                                tail_rows)])

    return k(xlin, wfull, nbr, cent)


def _final_body(p0_ref, p1_ref, nb_ref, wpost_ref, wself_ref, o_ref):
    t = p0_ref[...] + p1_ref[...]
    y = (jnp.dot(t, wpost_ref[...], preferred_element_type=jnp.float32)
         + jnp.dot(nb_ref[...], wself_ref[...],
                   preferred_element_type=jnp.float32))
    o_ref[...] = _silu(y)


def _tc_final(p0, p1, node, w_post, w_self, bm):
    n, d = node.shape
    full = lambda i: (0, 0)
    blk = pl.BlockSpec((bm, d), lambda i: (i, 0))
    return pl.pallas_call(
        _final_body,
        grid=(n // bm,),
        in_specs=[blk, blk, blk,
                  pl.BlockSpec((d, d), full),
                  pl.BlockSpec((d, d), full)],
        out_specs=blk,
        out_shape=jax.ShapeDtypeStruct((n, d), jnp.float32),
    )(p0, p1, node, w_post, w_self)


def kernel(node_embeddings, Z_embeddings, neighbour_distances, edge_embedding,
           graph, bessel_freqs, mlp_w1, mlp_b1, mlp_w2, mlp_b2, mlp_w3,
           mlp_b3, w_pre, w_post, w_self):
    n, d = node_embeddings.shape
    cent = graph[0]
    nbr = graph[1]
    x_lin = _tc_xlin(node_embeddings, w_pre, bm=400)
    wfull = _tc_wgen(neighbour_distances, edge_embedding, bessel_freqs,
                     mlp_w1, mlp_b1, mlp_w2, mlp_b2, mlp_w3, mlp_b3, bm=20)
    partials = _sc_message(x_lin, wfull, nbr, cent, n)
    return _tc_final(partials[0], partials[1], node_embeddings,
                     w_post, w_self, bm=400)


# speedup vs baseline: 2.7425x; 1.3566x over previous
import functools

import jax
import jax.numpy as jnp
from jax import lax
from jax.experimental import pallas as pl
from jax.experimental.pallas import tpu as pltpu
from jax.experimental.pallas import tpu_sc as plsc

_CUTOFF = 5.0


def _silu(x):
    return x / (1.0 + jnp.exp(-x))


def _xlin_body(x_ref, w_ref, o_ref):
    o_ref[...] = jnp.dot(x_ref[...], w_ref[...],
                         preferred_element_type=jnp.float32)


def _tc_xlin(node, w_pre, bm):
    n, d = node.shape
    return pl.pallas_call(
        _xlin_body,
        grid=(n // bm,),
        in_specs=[
            pl.BlockSpec((bm, d), lambda i: (i, 0)),
            pl.BlockSpec((d, d), lambda i: (0, 0)),
        ],
        out_specs=pl.BlockSpec((bm, d), lambda i: (i, 0)),
        out_shape=jax.ShapeDtypeStruct((n, d), jnp.float32),
    )(node, w_pre)


def _wgen_body(bm, rb, r_ref, ee_ref, fr_ref, w1_ref, b1_ref, w2_ref,
               b2_ref, w3_ref, b3_ref, o_ref):
    nl = 128
    r = r_ref[0]
    r3 = r[:, None, :]
    fr = fr_ref[...][None, :, None]
    bes = jnp.sin(r3 * fr) * (jnp.sqrt(2.0 / _CUTOFF) / r3)
    u = r / _CUTOFF
    u2 = u * u
    u6 = u2 * u2 * u2
    env = 1.0 - 28.0 * u6 + 48.0 * u6 * u - 21.0 * u6 * u2
    env = jnp.where(u < 1.0, env, 0.0)
    scale = env * ee_ref[0]
    aug = jnp.concatenate([bes, scale[:, None, :]], axis=1)
    t = jnp.swapaxes(aug, 1, 2).reshape(bm * nl, rb + 1)
    x = t[:, :rb]
    sc = t[:, rb:]
    h = _silu(jnp.dot(x, w1_ref[...],
                      preferred_element_type=jnp.float32) + b1_ref[...])
    h = _silu(jnp.dot(h, w2_ref[...],
                      preferred_element_type=jnp.float32) + b2_ref[...])
    g = h * sc
    o_ref[...] = (jnp.dot(g, w3_ref[...], preferred_element_type=jnp.float32)
                  + jnp.dot(sc, b3_ref[...],
                            preferred_element_type=jnp.float32))


def _tc_wgen(dist, ee, freqs, w1, b1, w2, b2, w3, b3, bm):
    e = dist.shape[0]
    rb = freqs.shape[0]
    d = w3.shape[1]
    nl = 128
    nblk = e // (nl * bm)
    r2 = dist.reshape(nblk, bm, nl)
    ee2 = ee.reshape(nblk, bm, nl)
    full = lambda i: (0, 0)
    return pl.pallas_call(
        functools.partial(_wgen_body, bm, rb),
        grid=(nblk,),
        in_specs=[
            pl.BlockSpec((1, bm, nl), lambda i: (i, 0, 0)),
            pl.BlockSpec((1, bm, nl), lambda i: (i, 0, 0)),
            pl.BlockSpec((rb,), lambda i: (0,)),
            pl.BlockSpec((rb, rb), full),
            pl.BlockSpec((1, rb), full),
            pl.BlockSpec((rb, rb), full),
            pl.BlockSpec((1, rb), full),
            pl.BlockSpec((rb, d), full),
            pl.BlockSpec((1, d), full),
        ],
        out_specs=pl.BlockSpec((bm * nl, d), lambda i: (i, 0)),
        out_shape=jax.ShapeDtypeStruct((e, d), jnp.float32),
    )(r2, ee2, freqs, w1, b1.reshape(1, rb), w2, b2.reshape(1, rb),
      w3, b3.reshape(1, d))


def _sc_message(xlin, wfull, nbr, cent, n_nodes):
    e, d = wfull.shape
    ncores, nsub = 2, 16
    nworkers = ncores * nsub
    epw = e // nworkers
    chunk = 80
    nchunks = epw // chunk
    rows_per_tile = (n_nodes // nsub) // 8 * 8
    tail_rows = n_nodes - rows_per_tile * nsub
    nz_full = rows_per_tile // chunk
    nz_rem = rows_per_tile - nz_full * chunk

    mesh = plsc.VectorSubcoreMesh(core_axis_name="c", subcore_axis_name="s")

    @functools.partial(
        pl.kernel,
        mesh=mesh,
        out_type=jax.ShapeDtypeStruct((ncores, n_nodes, d), jnp.float32),
        scratch_types=[
            pltpu.VMEM((chunk,), jnp.int32),
            pltpu.VMEM((chunk,), jnp.int32),
            pltpu.VMEM((chunk, d), jnp.float32),
            pltpu.VMEM((chunk, d), jnp.float32),
            pltpu.VMEM_SHARED((n_nodes, d), jnp.float32),
            pltpu.SemaphoreType.DMA,
        ],
    )
    def k(xlin_hbm, w_hbm, nbr_hbm, cent_hbm, out_hbm,
          nbr_v, cent_v, rows_v, msg_v, acc, sem):
        c = lax.axis_index("c")
        s = lax.axis_index("s")
        wid = c * nsub + s

        zero = jnp.zeros((16,), jnp.float32)

        def zrow(i, carry):
            for j in range(d // 16):
                msg_v[i, pl.ds(j * 16, 16)] = zero
            return carry

        lax.fori_loop(0, chunk, zrow, 0)
        base_row = s * rows_per_tile
        for rblk in range(nz_full):
            pltpu.sync_copy(msg_v, acc.at[pl.ds(base_row + rblk * chunk,
                                                chunk)])
        if nz_rem:
            pltpu.sync_copy(msg_v.at[pl.ds(0, nz_rem)],
                            acc.at[pl.ds(base_row + nz_full * chunk, nz_rem)])

        @pl.when(s == nsub - 1)
        def _zero_tail():
            pltpu.sync_copy(msg_v.at[pl.ds(0, tail_rows)],
                            acc.at[pl.ds(rows_per_tile * nsub, tail_rows)])

        plsc.subcore_barrier()

        ebase = wid * epw

        def body(i, carry):
            off = ebase + i * chunk
            pltpu.sync_copy(nbr_hbm.at[pl.ds(off, chunk)], nbr_v)
            pltpu.sync_copy(cent_hbm.at[pl.ds(off, chunk)], cent_v)
            pltpu.async_copy(xlin_hbm.at[nbr_v], rows_v, sem).wait()
            pltpu.sync_copy(w_hbm.at[pl.ds(off, chunk)], msg_v)

            def mrow(r, carry2):
                for j in range(d // 16):
                    sl = pl.ds(j * 16, 16)
                    msg_v[r, sl] = msg_v[r, sl] * rows_v[r, sl]
                return carry2

            lax.fori_loop(0, chunk, mrow, 0)
            pltpu.sync_copy(msg_v, acc.at[cent_v], add=True)
            return carry

        lax.fori_loop(0, nchunks, body, 0)
        plsc.subcore_barrier()
        pltpu.sync_copy(acc.at[pl.ds(base_row, rows_per_tile)],
                        out_hbm.at[c, pl.ds(base_row, rows_per_tile)])

        @pl.when(s == nsub - 1)
        def _write_tail():
            pltpu.sync_copy(acc.at[pl.ds(rows_per_tile * nsub, tail_rows)],
                            out_hbm.at[c, pl.ds(rows_per_tile * nsub,
                                                tail_rows)])

    return k(xlin, wfull, nbr, cent)


def _final_body(p0_ref, p1_ref, nb_ref, wpost_ref, wself_ref, o_ref):
    t = p0_ref[...] + p1_ref[...]
    y = (jnp.dot(t, wpost_ref[...], preferred_element_type=jnp.float32)
         + jnp.dot(nb_ref[...], wself_ref[...],
                   preferred_element_type=jnp.float32))
    o_ref[...] = _silu(y)


def _tc_final(p0, p1, node, w_post, w_self, bm):
    n, d = node.shape
    full = lambda i: (0, 0)
    blk = pl.BlockSpec((bm, d), lambda i: (i, 0))
    return pl.pallas_call(
        _final_body,
        grid=(n // bm,),
        in_specs=[blk, blk, blk,
                  pl.BlockSpec((d, d), full),
                  pl.BlockSpec((d, d), full)],
        out_specs=blk,
        out_shape=jax.ShapeDtypeStruct((n, d), jnp.float32),
    )(p0, p1, node, w_post, w_self)


def kernel(node_embeddings, Z_embeddings, neighbour_distances, edge_embedding,
           graph, bessel_freqs, mlp_w1, mlp_b1, mlp_w2, mlp_b2, mlp_w3,
           mlp_b3, w_pre, w_post, w_self):
    n, d = node_embeddings.shape
    cent = graph[0]
    nbr = graph[1]
    x_lin = _tc_xlin(node_embeddings, w_pre, bm=400)
    wfull = _tc_wgen(neighbour_distances, edge_embedding, bessel_freqs,
                     mlp_w1, mlp_b1, mlp_w2, mlp_b2, mlp_w3, mlp_b3, bm=20)
    partials = _sc_message(x_lin, wfull, nbr, cent, n)
    return _tc_final(partials[0], partials[1], node_embeddings,
                     w_post, w_self, bm=400)
